# baseline (speedup 1.0000x reference)
import functools

import jax
import jax.numpy as jnp
from jax import lax
from jax.experimental import pallas as pl
from jax.experimental.pallas import tpu as pltpu
from jax.experimental.pallas import tpu_sc as plsc

N = 10000
E = 320000
NC = 2
NS = 16
NW = NC * NS
NPAD = 10240
SLAB = NPAD // NS
EPW = E // NW
C = 80
NCHUNK = EPW // C
C3 = 2000
R1 = 144
R2 = 80

_f32 = jnp.float32


def _tc1_body(x_ref, w1_ref, al_ref, ar_ref, srcdata_ref, ert_ref):
    feat = jnp.dot(x_ref[...], w1_ref[...], preferred_element_type=_f32)
    el = jnp.dot(feat, al_ref[...], preferred_element_type=_f32)
    er = jnp.dot(feat, ar_ref[...], preferred_element_type=_f32)
    srcdata_ref[:, 0:16] = el
    srcdata_ref[:, 16:R1] = feat
    ert_ref[...] = er


def _tc2_body(acc_ref, b1_ref, rexp_ref, w2_ref, al2_ref, ar2_ref,
              srcdata2_ref, er2t_ref):
    acc = acc_ref[0] + acc_ref[1]
    dn = jnp.dot(acc[:, 0:16], rexp_ref[...], preferred_element_type=_f32)
    x = jnp.maximum(acc[:, 16:R1] / (dn + 1e-9) + b1_ref[...], 0.0)
    feat2 = jnp.dot(x, w2_ref[...], preferred_element_type=_f32)
    el2 = jnp.dot(feat2, al2_ref[...], preferred_element_type=_f32)
    er2 = jnp.dot(feat2, ar2_ref[...], preferred_element_type=_f32)
    srcdata2_ref[:, 0:16] = el2
    srcdata2_ref[:, 16:R2] = feat2
    er2t_ref[...] = er2


def _tc3_body(acc_ref, b2_ref, rexp2_ref, wp16_ref, ptab_ref):
    acc = acc_ref[0] + acc_ref[1]
    dn = jnp.dot(acc[:, 0:16], rexp2_ref[...], preferred_element_type=_f32)
    x2 = jnp.maximum(acc[:, 16:R2] / (dn + 1e-9) + b2_ref[...], 0.0)
    ptab_ref[...] = jnp.dot(x2, wp16_ref[...], preferred_element_type=_f32)


_BN = 1000
_GRID = N // _BN


def _tc1(features, W1, Al16, Ar16):
    return pl.pallas_call(
        _tc1_body,
        grid=(_GRID,),
        in_specs=[
            pl.BlockSpec((_BN, 128), lambda i: (i, 0)),
            pl.BlockSpec((128, 128), lambda i: (0, 0)),
            pl.BlockSpec((128, 16), lambda i: (0, 0)),
            pl.BlockSpec((128, 16), lambda i: (0, 0)),
        ],
        out_specs=[
            pl.BlockSpec((_BN, R1), lambda i: (i, 0)),
            pl.BlockSpec((_BN, 16), lambda i: (i, 0)),
        ],
        out_shape=[
            jax.ShapeDtypeStruct((N, R1), _f32),
            jax.ShapeDtypeStruct((N, 16), _f32),
        ],
    )(features, W1, Al16, Ar16)


def _tc2(acc1, b1r, Rexp, W2, Al2, Ar2):
    return pl.pallas_call(
        _tc2_body,
        grid=(_GRID,),
        in_specs=[
            pl.BlockSpec((2, _BN, R1), lambda i: (0, i, 0)),
            pl.BlockSpec((1, 128), lambda i: (0, 0)),
            pl.BlockSpec((16, 128), lambda i: (0, 0)),
            pl.BlockSpec((128, 64), lambda i: (0, 0)),
            pl.BlockSpec((64, 16), lambda i: (0, 0)),
            pl.BlockSpec((64, 16), lambda i: (0, 0)),
        ],
        out_specs=[
            pl.BlockSpec((_BN, R2), lambda i: (i, 0)),
            pl.BlockSpec((_BN, 16), lambda i: (i, 0)),
        ],
        out_shape=[
            jax.ShapeDtypeStruct((N, R2), _f32),
            jax.ShapeDtypeStruct((N, 16), _f32),
        ],
    )(acc1, b1r, Rexp, W2, Al2, Ar2)


def _tc3(acc2, b2r, Rexp2, Wp16):
    return pl.pallas_call(
        _tc3_body,
        grid=(_GRID,),
        in_specs=[
            pl.BlockSpec((2, _BN, R2), lambda i: (0, i, 0)),
            pl.BlockSpec((1, 64), lambda i: (0, 0)),
            pl.BlockSpec((16, 64), lambda i: (0, 0)),
            pl.BlockSpec((64, 16), lambda i: (0, 0)),
        ],
        out_specs=pl.BlockSpec((_BN, 16), lambda i: (i, 0)),
        out_shape=jax.ShapeDtypeStruct((N, 16), _f32),
    )(acc2, b2r, Rexp2, Wp16)


BLK = 25
NBLK = NCHUNK // BLK


def _make_sc_conv(R, nhchunks, per_head, nbuf):
    mesh = plsc.VectorSubcoreMesh(core_axis_name="c", subcore_axis_name="s")

    @functools.partial(
        pl.kernel,
        out_type=jax.ShapeDtypeStruct((NC, NPAD, R), _f32),
        mesh=mesh,
        compiler_params=pltpu.CompilerParams(use_tc_tiling_on_sc=False),
        scratch_types=[
            pltpu.VMEM_SHARED((NPAD, R), _f32),
            pltpu.VMEM((BLK, C), jnp.int32),
            pltpu.VMEM((BLK, C), jnp.int32),
        ]
        + [pltpu.VMEM((C, R), _f32) for _ in range(nbuf)]
        + [pltpu.VMEM((C, 16), _f32) for _ in range(nbuf)]
        + [pltpu.SemaphoreType.DMA for _ in range(3 * nbuf)],
    )
    def sc_conv(srcdata_hbm, ert_hbm, src3_hbm, dst3_hbm, zrow_hbm, out_hbm,
                accum, src_buf, dst_buf, *bufs):
        rows = bufs[0:nbuf]
        erb = bufs[nbuf:2 * nbuf]
        rsem = bufs[2 * nbuf:3 * nbuf]
        esem = bufs[3 * nbuf:4 * nbuf]
        ssem = bufs[4 * nbuf:5 * nbuf]
        c = lax.axis_index("c")
        s = lax.axis_index("s")
        wid = s * NC + c
        slab = s * SLAB
        pltpu.sync_copy(zrow_hbm, rows[0])
        for k in range(SLAB // C):
            pltpu.sync_copy(rows[0], accum.at[pl.ds(slab + k * C, C)])
        plsc.subcore_barrier()

        def issue(kc, b):
            pltpu.async_copy(srcdata_hbm.at[src_buf.at[kc]], rows[b], rsem[b])
            pltpu.async_copy(ert_hbm.at[dst_buf.at[kc]], erb[b], esem[b])

        def wait_scatter(kc, b):
            pltpu.make_async_copy(rows[b], accum.at[dst_buf.at[kc]],
                                  ssem[b]).wait()

        def process(kc, b):
            pltpu.make_async_copy(srcdata_hbm.at[src_buf.at[kc]],
                                  rows[b], rsem[b]).wait()
            pltpu.make_async_copy(ert_hbm.at[dst_buf.at[kc]],
                                  erb[b], esem[b]).wait()

            def edge(i, cy):
                v = rows[b][i, 0:16] + erb[b][i, 0:16]
                t = jnp.exp(jnp.maximum(v, 0.2 * v))
                rows[b][i, 0:16] = t
                for h in range(nhchunks):
                    g = t[h if per_head else 0]
                    sl = pl.ds(16 + 16 * h, 16)
                    rows[b][i, sl] = rows[b][i, sl] * g
                return cy

            lax.fori_loop(0, C, edge, 0)
            pltpu.async_copy(rows[b], accum.at[dst_buf.at[kc]], ssem[b],
                             add=True)

        def block(bi, carry):
            pltpu.sync_copy(src3_hbm.at[wid, pl.ds(bi * BLK, BLK)], src_buf)
            pltpu.sync_copy(dst3_hbm.at[wid, pl.ds(bi * BLK, BLK)], dst_buf)
            for b in range(nbuf):
                issue(b, b)
            for kc in range(BLK):
                process(kc, kc % nbuf)
                j = kc + nbuf - 1
                if kc >= 1 and j < BLK:
                    wait_scatter(kc - 1, (kc - 1) % nbuf)
                    issue(j, (kc - 1) % nbuf)
            for kc in range(BLK - nbuf, BLK):
                wait_scatter(kc, kc % nbuf)
            return carry

        lax.fori_loop(0, NBLK, block, 0)

        plsc.subcore_barrier()
        pltpu.sync_copy(accum.at[pl.ds(slab, SLAB)],
                        out_hbm.at[c, pl.ds(slab, SLAB)])

    return sc_conv


def _make_sc_final():
    mesh = plsc.VectorSubcoreMesh(core_axis_name="c", subcore_axis_name="s")

    @functools.partial(
        pl.kernel,
        out_type=jax.ShapeDtypeStruct((E,), _f32),
        mesh=mesh,
        compiler_params=pltpu.CompilerParams(
            use_tc_tiling_on_sc=False, needs_layout_passes=False),
        scratch_types=[
            pltpu.VMEM((N,), _f32),
            pltpu.VMEM((N,), _f32),
            pltpu.VMEM((C3,), jnp.int32),
            pltpu.VMEM((C3,), jnp.int32),
            pltpu.VMEM((C3,), _f32),
        ],
    )
    def sc_final(psrc_hbm, pdst_hbm, src_hbm, dst_hbm, out_hbm,
                 pst, pdt, idxs, idxd, ob):
        c = lax.axis_index("c")
        s = lax.axis_index("s")
        wid = s * NC + c
        pltpu.sync_copy(psrc_hbm, pst)
        pltpu.sync_copy(pdst_hbm, pdt)
        ebase = wid * EPW

        def chunk(kc, carry):
            base = ebase + kc * C3
            pltpu.sync_copy(src_hbm.at[pl.ds(base, C3)], idxs)
            pltpu.sync_copy(dst_hbm.at[pl.ds(base, C3)], idxd)

            def grp(j, cy):
                a = plsc.load_gather(pst, [idxs[pl.ds(j * 16, 16)]])
                b = plsc.load_gather(pdt, [idxd[pl.ds(j * 16, 16)]])
                z = a + b
                ob[pl.ds(j * 16, 16)] = 1.0 / (1.0 + jnp.exp(-z))
                return cy

            lax.fori_loop(0, C3 // 16, grp, 0)
            pltpu.sync_copy(ob, out_hbm.at[pl.ds(base, C3)])
            return carry

        lax.fori_loop(0, EPW // C3, chunk, 0)

    return sc_final


def kernel(features, edge_index, edge_type, W1, attn_l1, attn_r1, b1,
           W2, attn_l2, attn_r2, b2, Wp, bp):
    del edge_type
    src = edge_index[0]
    dst = edge_index[1]

    idx128 = jnp.arange(128)
    Al16 = jnp.zeros((128, 16), _f32).at[idx128, idx128 // 16].set(
        attn_l1.reshape(128))
    Ar16 = jnp.zeros((128, 16), _f32).at[idx128, idx128 // 16].set(
        attn_r1.reshape(128))
    Rexp = (jnp.arange(16)[:, None] == (idx128[None, :] // 16)).astype(_f32)
    Al2 = jnp.zeros((64, 16), _f32).at[:, 0].set(attn_l2[0])
    Ar2 = jnp.zeros((64, 16), _f32).at[:, 0].set(attn_r2[0])
    Rexp2 = jnp.zeros((16, 64), _f32).at[0, :].set(1.0)
    Wp16 = jnp.zeros((64, 16), _f32).at[:, 0].set(Wp[:64, 0]).at[:, 1].set(
        Wp[64:, 0])
    b1r = b1.reshape(1, 128)
    b2r = b2.reshape(1, 64)

    srcdata1, er1t = _tc1(features, W1, Al16, Ar16)

    src3 = src.reshape(NW, NCHUNK, C)
    dst3 = dst.reshape(NW, NCHUNK, C)
    zrow1 = jnp.zeros((C, R1), _f32)
    acc1 = _make_sc_conv(R1, 8, True, 2)(srcdata1, er1t, src3, dst3, zrow1)

    srcdata2, er2t = _tc2(acc1, b1r, Rexp, W2, Al2, Ar2)

    zrow2 = jnp.zeros((C, R2), _f32)
    acc2 = _make_sc_conv(R2, 4, False, 4)(srcdata2, er2t, src3, dst3, zrow2)

    ptab = _tc3(acc2, b2r, Rexp2, Wp16)
    psrc = ptab[:, 0] + bp[0]
    pdst = ptab[:, 1]

    return _make_sc_final()(psrc, pdst, src, dst)


# speedup vs baseline: 1.1466x; 1.1466x over previous
import functools

import jax
import jax.numpy as jnp
from jax import lax
from jax.experimental import pallas as pl
from jax.experimental.pallas import tpu as pltpu
from jax.experimental.pallas import tpu_sc as plsc

N = 10000
E = 320000
NC = 2
NS = 16
NW = NC * NS
NPAD = 10240
SLAB = NPAD // NS
EPW = E // NW
C = 80
NCHUNK = EPW // C
C3 = 2000
R1 = 144
R2 = 80

_f32 = jnp.float32


def _tc1_body(x_ref, w1_ref, al_ref, ar_ref, srcdata_ref, ert_ref):
    feat = jnp.dot(x_ref[...], w1_ref[...], preferred_element_type=_f32)
    el = jnp.dot(feat, al_ref[...], preferred_element_type=_f32)
    er = jnp.dot(feat, ar_ref[...], preferred_element_type=_f32)
    srcdata_ref[:, 0:16] = el
    srcdata_ref[:, 16:R1] = feat
    ert_ref[...] = er


def _tc2_body(acc_ref, b1_ref, rexp_ref, w2_ref, al2_ref, ar2_ref,
              srcdata2_ref, er2t_ref):
    acc = acc_ref[0] + acc_ref[1]
    dn = jnp.dot(acc[:, 0:16], rexp_ref[...], preferred_element_type=_f32)
    x = jnp.maximum(acc[:, 16:R1] / (dn + 1e-9) + b1_ref[...], 0.0)
    feat2 = jnp.dot(x, w2_ref[...], preferred_element_type=_f32)
    el2 = jnp.dot(feat2, al2_ref[...], preferred_element_type=_f32)
    er2 = jnp.dot(feat2, ar2_ref[...], preferred_element_type=_f32)
    srcdata2_ref[:, 0:16] = el2
    srcdata2_ref[:, 16:R2] = feat2
    er2t_ref[...] = er2


def _tc3_body(acc_ref, b2_ref, rexp2_ref, wp16_ref, ptab_ref):
    acc = acc_ref[0] + acc_ref[1]
    dn = jnp.dot(acc[:, 0:16], rexp2_ref[...], preferred_element_type=_f32)
    x2 = jnp.maximum(acc[:, 16:R2] / (dn + 1e-9) + b2_ref[...], 0.0)
    ptab_ref[...] = jnp.dot(x2, wp16_ref[...], preferred_element_type=_f32)


_BN = 1000
_GRID = N // _BN


def _tc1(features, W1, Al16, Ar16):
    return pl.pallas_call(
        _tc1_body,
        grid=(_GRID,),
        in_specs=[
            pl.BlockSpec((_BN, 128), lambda i: (i, 0)),
            pl.BlockSpec((128, 128), lambda i: (0, 0)),
            pl.BlockSpec((128, 16), lambda i: (0, 0)),
            pl.BlockSpec((128, 16), lambda i: (0, 0)),
        ],
        out_specs=[
            pl.BlockSpec((_BN, R1), lambda i: (i, 0)),
            pl.BlockSpec((_BN, 16), lambda i: (i, 0)),
        ],
        out_shape=[
            jax.ShapeDtypeStruct((N, R1), _f32),
            jax.ShapeDtypeStruct((N, 16), _f32),
        ],
    )(features, W1, Al16, Ar16)


def _tc2(acc1, b1r, Rexp, W2, Al2, Ar2):
    return pl.pallas_call(
        _tc2_body,
        grid=(_GRID,),
        in_specs=[
            pl.BlockSpec((2, _BN, R1), lambda i: (0, i, 0)),
            pl.BlockSpec((1, 128), lambda i: (0, 0)),
            pl.BlockSpec((16, 128), lambda i: (0, 0)),
            pl.BlockSpec((128, 64), lambda i: (0, 0)),
            pl.BlockSpec((64, 16), lambda i: (0, 0)),
            pl.BlockSpec((64, 16), lambda i: (0, 0)),
        ],
        out_specs=[
            pl.BlockSpec((_BN, R2), lambda i: (i, 0)),
            pl.BlockSpec((_BN, 16), lambda i: (i, 0)),
        ],
        out_shape=[
            jax.ShapeDtypeStruct((N, R2), _f32),
            jax.ShapeDtypeStruct((N, 16), _f32),
        ],
    )(acc1, b1r, Rexp, W2, Al2, Ar2)


def _tc3(acc2, b2r, Rexp2, Wp16):
    return pl.pallas_call(
        _tc3_body,
        grid=(_GRID,),
        in_specs=[
            pl.BlockSpec((2, _BN, R2), lambda i: (0, i, 0)),
            pl.BlockSpec((1, 64), lambda i: (0, 0)),
            pl.BlockSpec((16, 64), lambda i: (0, 0)),
            pl.BlockSpec((64, 16), lambda i: (0, 0)),
        ],
        out_specs=pl.BlockSpec((_BN, 16), lambda i: (i, 0)),
        out_shape=jax.ShapeDtypeStruct((N, 16), _f32),
    )(acc2, b2r, Rexp2, Wp16)


BLK = 25


def _make_sc_conv(R, nhchunks, per_head, nbuf, ec):
    nchunk = EPW // ec
    nblk = nchunk // BLK
    mesh = plsc.VectorSubcoreMesh(core_axis_name="c", subcore_axis_name="s")

    @functools.partial(
        pl.kernel,
        out_type=jax.ShapeDtypeStruct((NC, NPAD, R), _f32),
        mesh=mesh,
        compiler_params=pltpu.CompilerParams(use_tc_tiling_on_sc=False),
        scratch_types=[
            pltpu.VMEM_SHARED((NPAD, R), _f32),
            pltpu.VMEM((BLK, ec), jnp.int32),
            pltpu.VMEM((BLK, ec), jnp.int32),
        ]
        + [pltpu.VMEM((ec, R), _f32) for _ in range(nbuf)]
        + [pltpu.VMEM((ec, 16), _f32) for _ in range(nbuf)]
        + [pltpu.SemaphoreType.DMA for _ in range(3 * nbuf)],
    )
    def sc_conv(srcdata_hbm, ert_hbm, src3_hbm, dst3_hbm, zrow_hbm, out_hbm,
                accum, src_buf, dst_buf, *bufs):
        rows = bufs[0:nbuf]
        erb = bufs[nbuf:2 * nbuf]
        rsem = bufs[2 * nbuf:3 * nbuf]
        esem = bufs[3 * nbuf:4 * nbuf]
        ssem = bufs[4 * nbuf:5 * nbuf]
        c = lax.axis_index("c")
        s = lax.axis_index("s")
        wid = s * NC + c
        slab = s * SLAB
        pltpu.sync_copy(zrow_hbm, rows[0])
        for k in range(SLAB // ec):
            pltpu.sync_copy(rows[0], accum.at[pl.ds(slab + k * ec, ec)])
        rem = SLAB % ec
        if rem:
            pltpu.sync_copy(rows[0].at[pl.ds(0, rem)],
                            accum.at[pl.ds(slab + (SLAB // ec) * ec, rem)])
        plsc.subcore_barrier()

        def issue(kc, b):
            pltpu.async_copy(srcdata_hbm.at[src_buf.at[kc]], rows[b], rsem[b])
            pltpu.async_copy(ert_hbm.at[dst_buf.at[kc]], erb[b], esem[b])

        def wait_scatter(kc, b):
            pltpu.make_async_copy(rows[b], accum.at[dst_buf.at[kc]],
                                  ssem[b]).wait()

        def process(kc, b):
            pltpu.make_async_copy(srcdata_hbm.at[src_buf.at[kc]],
                                  rows[b], rsem[b]).wait()
            pltpu.make_async_copy(ert_hbm.at[dst_buf.at[kc]],
                                  erb[b], esem[b]).wait()

            def edge(i, cy):
                v = rows[b][i, 0:16] + erb[b][i, 0:16]
                t = jnp.exp(jnp.maximum(v, 0.2 * v))
                rows[b][i, 0:16] = t
                for h in range(nhchunks):
                    g = t[h if per_head else 0]
                    sl = pl.ds(16 + 16 * h, 16)
                    rows[b][i, sl] = rows[b][i, sl] * g
                return cy

            lax.fori_loop(0, ec, edge, 0)
            pltpu.async_copy(rows[b], accum.at[dst_buf.at[kc]], ssem[b],
                             add=True)

        def block(bi, carry):
            pltpu.sync_copy(src3_hbm.at[wid, pl.ds(bi * BLK, BLK)], src_buf)
            pltpu.sync_copy(dst3_hbm.at[wid, pl.ds(bi * BLK, BLK)], dst_buf)
            for b in range(nbuf):
                issue(b, b)
            for kc in range(BLK):
                process(kc, kc % nbuf)
                j = kc + nbuf - 1
                if kc >= 1 and j < BLK:
                    wait_scatter(kc - 1, (kc - 1) % nbuf)
                    issue(j, (kc - 1) % nbuf)
            for kc in range(BLK - nbuf, BLK):
                wait_scatter(kc, kc % nbuf)
            return carry

        lax.fori_loop(0, nblk, block, 0)

        plsc.subcore_barrier()
        pltpu.sync_copy(accum.at[pl.ds(slab, SLAB)],
                        out_hbm.at[c, pl.ds(slab, SLAB)])

    return sc_conv


def _make_sc_final():
    mesh = plsc.VectorSubcoreMesh(core_axis_name="c", subcore_axis_name="s")

    @functools.partial(
        pl.kernel,
        out_type=jax.ShapeDtypeStruct((E,), _f32),
        mesh=mesh,
        compiler_params=pltpu.CompilerParams(
            use_tc_tiling_on_sc=False, needs_layout_passes=False),
        scratch_types=[
            pltpu.VMEM((N,), _f32),
            pltpu.VMEM((N,), _f32),
            pltpu.VMEM((C3,), jnp.int32),
            pltpu.VMEM((C3,), jnp.int32),
            pltpu.VMEM((C3,), _f32),
        ],
    )
    def sc_final(psrc_hbm, pdst_hbm, src_hbm, dst_hbm, out_hbm,
                 pst, pdt, idxs, idxd, ob):
        c = lax.axis_index("c")
        s = lax.axis_index("s")
        wid = s * NC + c
        pltpu.sync_copy(psrc_hbm, pst)
        pltpu.sync_copy(pdst_hbm, pdt)
        ebase = wid * EPW

        def chunk(kc, carry):
            base = ebase + kc * C3
            pltpu.sync_copy(src_hbm.at[pl.ds(base, C3)], idxs)
            pltpu.sync_copy(dst_hbm.at[pl.ds(base, C3)], idxd)

            def grp(j, cy):
                a = plsc.load_gather(pst, [idxs[pl.ds(j * 16, 16)]])
                b = plsc.load_gather(pdt, [idxd[pl.ds(j * 16, 16)]])
                z = a + b
                ob[pl.ds(j * 16, 16)] = 1.0 / (1.0 + jnp.exp(-z))
                return cy

            lax.fori_loop(0, C3 // 16, grp, 0)
            pltpu.sync_copy(ob, out_hbm.at[pl.ds(base, C3)])
            return carry

        lax.fori_loop(0, EPW // C3, chunk, 0)

    return sc_final


def kernel(features, edge_index, edge_type, W1, attn_l1, attn_r1, b1,
           W2, attn_l2, attn_r2, b2, Wp, bp):
    del edge_type
    src = edge_index[0]
    dst = edge_index[1]

    idx128 = jnp.arange(128)
    Al16 = jnp.zeros((128, 16), _f32).at[idx128, idx128 // 16].set(
        attn_l1.reshape(128))
    Ar16 = jnp.zeros((128, 16), _f32).at[idx128, idx128 // 16].set(
        attn_r1.reshape(128))
    Rexp = (jnp.arange(16)[:, None] == (idx128[None, :] // 16)).astype(_f32)
    Al2 = jnp.zeros((64, 16), _f32).at[:, 0].set(attn_l2[0])
    Ar2 = jnp.zeros((64, 16), _f32).at[:, 0].set(attn_r2[0])
    Rexp2 = jnp.zeros((16, 64), _f32).at[0, :].set(1.0)
    Wp16 = jnp.zeros((64, 16), _f32).at[:, 0].set(Wp[:64, 0]).at[:, 1].set(
        Wp[64:, 0])
    b1r = b1.reshape(1, 128)
    b2r = b2.reshape(1, 64)

    srcdata1, er1t = _tc1(features, W1, Al16, Ar16)

    c1 = 50
    src3a = src.reshape(NW, EPW // c1, c1)
    dst3a = dst.reshape(NW, EPW // c1, c1)
    zrow1 = jnp.zeros((c1, R1), _f32)
    acc1 = _make_sc_conv(R1, 8, True, 3, c1)(srcdata1, er1t, src3a, dst3a,
                                             zrow1)

    srcdata2, er2t = _tc2(acc1, b1r, Rexp, W2, Al2, Ar2)

    c2 = 80
    src3b = src.reshape(NW, EPW // c2, c2)
    dst3b = dst.reshape(NW, EPW // c2, c2)
    zrow2 = jnp.zeros((c2, R2), _f32)
    acc2 = _make_sc_conv(R2, 4, False, 4, c2)(srcdata2, er2t, src3b, dst3b,
                                              zrow2)

    ptab = _tc3(acc2, b2r, Rexp2, Wp16)
    psrc = ptab[:, 0] + bp[0]
    pdst = ptab[:, 1]

    return _make_sc_final()(psrc, pdst, src, dst)


# speedup vs baseline: 1.2142x; 1.0589x over previous
import functools

import jax
import jax.numpy as jnp
from jax import lax
from jax.experimental import pallas as pl
from jax.experimental.pallas import tpu as pltpu
from jax.experimental.pallas import tpu_sc as plsc

N = 10000
E = 320000
NC = 2
NS = 16
NW = NC * NS
NPAD = 10240
SLAB = NPAD // NS
EPW = E // NW
C = 80
NCHUNK = EPW // C
C3 = 2000
R1 = 144
R2 = 80

_f32 = jnp.float32


def _tc1_body(x_ref, w1_ref, al_ref, ar_ref, srcdata_ref, ert_ref):
    feat = jnp.dot(x_ref[...], w1_ref[...], preferred_element_type=_f32)
    el = jnp.dot(feat, al_ref[...], preferred_element_type=_f32)
    er = jnp.dot(feat, ar_ref[...], preferred_element_type=_f32)
    srcdata_ref[:, 0:16] = el
    srcdata_ref[:, 16:R1] = feat
    ert_ref[...] = er


def _tc2_body(acc_ref, b1_ref, rexp_ref, w2_ref, al2_ref, ar2_ref,
              srcdata2_ref, er2t_ref):
    acc = acc_ref[0] + acc_ref[1]
    dn = jnp.dot(acc[:, 0:16], rexp_ref[...], preferred_element_type=_f32)
    x = jnp.maximum(acc[:, 16:R1] / (dn + 1e-9) + b1_ref[...], 0.0)
    feat2 = jnp.dot(x, w2_ref[...], preferred_element_type=_f32)
    el2 = jnp.dot(feat2, al2_ref[...], preferred_element_type=_f32)
    er2 = jnp.dot(feat2, ar2_ref[...], preferred_element_type=_f32)
    srcdata2_ref[:, 0:16] = el2
    srcdata2_ref[:, 16:R2] = feat2
    er2t_ref[...] = er2


def _tc3_body(acc_ref, b2_ref, rexp2_ref, wp16_ref, ptab_ref):
    acc = acc_ref[0] + acc_ref[1]
    dn = jnp.dot(acc[:, 0:16], rexp2_ref[...], preferred_element_type=_f32)
    x2 = jnp.maximum(acc[:, 16:R2] / (dn + 1e-9) + b2_ref[...], 0.0)
    ptab_ref[...] = jnp.dot(x2, wp16_ref[...], preferred_element_type=_f32)


_BN = 1000
_GRID = N // _BN


def _tc1(features, W1, Al16, Ar16):
    return pl.pallas_call(
        _tc1_body,
        grid=(_GRID,),
        in_specs=[
            pl.BlockSpec((_BN, 128), lambda i: (i, 0)),
            pl.BlockSpec((128, 128), lambda i: (0, 0)),
            pl.BlockSpec((128, 16), lambda i: (0, 0)),
            pl.BlockSpec((128, 16), lambda i: (0, 0)),
        ],
        out_specs=[
            pl.BlockSpec((_BN, R1), lambda i: (i, 0)),
            pl.BlockSpec((_BN, 16), lambda i: (i, 0)),
        ],
        out_shape=[
            jax.ShapeDtypeStruct((N, R1), _f32),
            jax.ShapeDtypeStruct((N, 16), _f32),
        ],
    )(features, W1, Al16, Ar16)


def _tc2(acc1, b1r, Rexp, W2, Al2, Ar2):
    return pl.pallas_call(
        _tc2_body,
        grid=(_GRID,),
        in_specs=[
            pl.BlockSpec((2, _BN, R1), lambda i: (0, i, 0)),
            pl.BlockSpec((1, 128), lambda i: (0, 0)),
            pl.BlockSpec((16, 128), lambda i: (0, 0)),
            pl.BlockSpec((128, 64), lambda i: (0, 0)),
            pl.BlockSpec((64, 16), lambda i: (0, 0)),
            pl.BlockSpec((64, 16), lambda i: (0, 0)),
        ],
        out_specs=[
            pl.BlockSpec((_BN, R2), lambda i: (i, 0)),
            pl.BlockSpec((_BN, 16), lambda i: (i, 0)),
        ],
        out_shape=[
            jax.ShapeDtypeStruct((N, R2), _f32),
            jax.ShapeDtypeStruct((N, 16), _f32),
        ],
    )(acc1, b1r, Rexp, W2, Al2, Ar2)


def _tc3(acc2, b2r, Rexp2, Wp16):
    return pl.pallas_call(
        _tc3_body,
        grid=(_GRID,),
        in_specs=[
            pl.BlockSpec((2, _BN, R2), lambda i: (0, i, 0)),
            pl.BlockSpec((1, 64), lambda i: (0, 0)),
            pl.BlockSpec((16, 64), lambda i: (0, 0)),
            pl.BlockSpec((64, 16), lambda i: (0, 0)),
        ],
        out_specs=pl.BlockSpec((_BN, 16), lambda i: (i, 0)),
        out_shape=jax.ShapeDtypeStruct((N, 16), _f32),
    )(acc2, b2r, Rexp2, Wp16)


BLK = 25


def _make_sc_conv(R, nhchunks, per_head, nbuf, ec):
    nchunk = EPW // ec
    nblk = nchunk // BLK
    mesh = plsc.VectorSubcoreMesh(core_axis_name="c", subcore_axis_name="s")

    @functools.partial(
        pl.kernel,
        out_type=jax.ShapeDtypeStruct((NC, NPAD, R), _f32),
        mesh=mesh,
        compiler_params=pltpu.CompilerParams(use_tc_tiling_on_sc=False),
        scratch_types=[
            pltpu.VMEM_SHARED((NPAD, R), _f32),
            pltpu.VMEM((BLK, ec), jnp.int32),
            pltpu.VMEM((BLK, ec), jnp.int32),
        ]
        + [pltpu.VMEM((ec, R), _f32) for _ in range(nbuf)]
        + [pltpu.VMEM((ec, 16), _f32) for _ in range(nbuf)]
        + [pltpu.SemaphoreType.DMA for _ in range(3 * nbuf)],
    )
    def sc_conv(srcdata_hbm, ert_hbm, src3_hbm, dst3_hbm, zrow_hbm, out_hbm,
                accum, src_buf, dst_buf, *bufs):
        rows = bufs[0:nbuf]
        erb = bufs[nbuf:2 * nbuf]
        rsem = bufs[2 * nbuf:3 * nbuf]
        esem = bufs[3 * nbuf:4 * nbuf]
        ssem = bufs[4 * nbuf:5 * nbuf]
        c = lax.axis_index("c")
        s = lax.axis_index("s")
        wid = s * NC + c
        slab = s * SLAB
        pltpu.sync_copy(zrow_hbm, rows[0])
        for k in range(SLAB // ec):
            pltpu.sync_copy(rows[0], accum.at[pl.ds(slab + k * ec, ec)])
        rem = SLAB % ec
        if rem:
            pltpu.sync_copy(rows[0].at[pl.ds(0, rem)],
                            accum.at[pl.ds(slab + (SLAB // ec) * ec, rem)])
        plsc.subcore_barrier()

        def issue(kc, b):
            pltpu.async_copy(srcdata_hbm.at[src_buf.at[kc]], rows[b], rsem[b])
            pltpu.async_copy(ert_hbm.at[dst_buf.at[kc]], erb[b], esem[b])

        def wait_scatter(kc, b):
            pltpu.make_async_copy(rows[b], accum.at[dst_buf.at[kc]],
                                  ssem[b]).wait()

        def process(kc, b):
            pltpu.make_async_copy(srcdata_hbm.at[src_buf.at[kc]],
                                  rows[b], rsem[b]).wait()
            pltpu.make_async_copy(ert_hbm.at[dst_buf.at[kc]],
                                  erb[b], esem[b]).wait()

            def edge2(j, cy):
                i0 = j * 2
                ts = []
                for u in range(2):
                    i = i0 + u
                    v = rows[b][i, 0:16] + erb[b][i, 0:16]
                    t = jnp.exp(jnp.maximum(v, 0.2 * v))
                    rows[b][i, 0:16] = t
                    ts.append(t)
                for u in range(2):
                    i = i0 + u
                    t = ts[u]
                    for h in range(nhchunks):
                        g = t[h if per_head else 0]
                        sl = pl.ds(16 + 16 * h, 16)
                        rows[b][i, sl] = rows[b][i, sl] * g
                return cy

            lax.fori_loop(0, ec // 2, edge2, 0)
            pltpu.async_copy(rows[b], accum.at[dst_buf.at[kc]], ssem[b],
                             add=True)

        def block(bi, carry):
            pltpu.sync_copy(src3_hbm.at[wid, pl.ds(bi * BLK, BLK)], src_buf)
            pltpu.sync_copy(dst3_hbm.at[wid, pl.ds(bi * BLK, BLK)], dst_buf)
            for b in range(nbuf):
                issue(b, b)
            for kc in range(BLK):
                process(kc, kc % nbuf)
                j = kc + nbuf - 1
                if kc >= 1 and j < BLK:
                    wait_scatter(kc - 1, (kc - 1) % nbuf)
                    issue(j, (kc - 1) % nbuf)
            for kc in range(BLK - nbuf, BLK):
                wait_scatter(kc, kc % nbuf)
            return carry

        lax.fori_loop(0, nblk, block, 0)

        plsc.subcore_barrier()
        pltpu.sync_copy(accum.at[pl.ds(slab, SLAB)],
                        out_hbm.at[c, pl.ds(slab, SLAB)])

    return sc_conv


def _make_sc_final():
    mesh = plsc.VectorSubcoreMesh(core_axis_name="c", subcore_axis_name="s")

    @functools.partial(
        pl.kernel,
        out_type=jax.ShapeDtypeStruct((E,), _f32),
        mesh=mesh,
        compiler_params=pltpu.CompilerParams(
            use_tc_tiling_on_sc=False, needs_layout_passes=False),
        scratch_types=[
            pltpu.VMEM((N,), _f32),
            pltpu.VMEM((N,), _f32),
            pltpu.VMEM((C3,), jnp.int32),
            pltpu.VMEM((C3,), jnp.int32),
            pltpu.VMEM((C3,), _f32),
        ],
    )
    def sc_final(psrc_hbm, pdst_hbm, src_hbm, dst_hbm, out_hbm,
                 pst, pdt, idxs, idxd, ob):
        c = lax.axis_index("c")
        s = lax.axis_index("s")
        wid = s * NC + c
        pltpu.sync_copy(psrc_hbm, pst)
        pltpu.sync_copy(pdst_hbm, pdt)
        ebase = wid * EPW

        def chunk(kc, carry):
            base = ebase + kc * C3
            pltpu.sync_copy(src_hbm.at[pl.ds(base, C3)], idxs)
            pltpu.sync_copy(dst_hbm.at[pl.ds(base, C3)], idxd)

            def grp(j, cy):
                a = plsc.load_gather(pst, [idxs[pl.ds(j * 16, 16)]])
                b = plsc.load_gather(pdt, [idxd[pl.ds(j * 16, 16)]])
                z = a + b
                ob[pl.ds(j * 16, 16)] = 1.0 / (1.0 + jnp.exp(-z))
                return cy

            lax.fori_loop(0, C3 // 16, grp, 0)
            pltpu.sync_copy(ob, out_hbm.at[pl.ds(base, C3)])
            return carry

        lax.fori_loop(0, EPW // C3, chunk, 0)

    return sc_final


def kernel(features, edge_index, edge_type, W1, attn_l1, attn_r1, b1,
           W2, attn_l2, attn_r2, b2, Wp, bp):
    del edge_type
    src = edge_index[0]
    dst = edge_index[1]

    idx128 = jnp.arange(128)
    Al16 = jnp.zeros((128, 16), _f32).at[idx128, idx128 // 16].set(
        attn_l1.reshape(128))
    Ar16 = jnp.zeros((128, 16), _f32).at[idx128, idx128 // 16].set(
        attn_r1.reshape(128))
    Rexp = (jnp.arange(16)[:, None] == (idx128[None, :] // 16)).astype(_f32)
    Al2 = jnp.zeros((64, 16), _f32).at[:, 0].set(attn_l2[0])
    Ar2 = jnp.zeros((64, 16), _f32).at[:, 0].set(attn_r2[0])
    Rexp2 = jnp.zeros((16, 64), _f32).at[0, :].set(1.0)
    Wp16 = jnp.zeros((64, 16), _f32).at[:, 0].set(Wp[:64, 0]).at[:, 1].set(
        Wp[64:, 0])
    b1r = b1.reshape(1, 128)
    b2r = b2.reshape(1, 64)

    srcdata1, er1t = _tc1(features, W1, Al16, Ar16)

    c1 = 50
    src3a = src.reshape(NW, EPW // c1, c1)
    dst3a = dst.reshape(NW, EPW // c1, c1)
    zrow1 = jnp.zeros((c1, R1), _f32)
    acc1 = _make_sc_conv(R1, 8, True, 3, c1)(srcdata1, er1t, src3a, dst3a,
                                             zrow1)

    srcdata2, er2t = _tc2(acc1, b1r, Rexp, W2, Al2, Ar2)

    c2 = 80
    src3b = src.reshape(NW, EPW // c2, c2)
    dst3b = dst.reshape(NW, EPW // c2, c2)
    zrow2 = jnp.zeros((c2, R2), _f32)
    acc2 = _make_sc_conv(R2, 4, False, 4, c2)(srcdata2, er2t, src3b, dst3b,
                                              zrow2)

    ptab = _tc3(acc2, b2r, Rexp2, Wp16)
    psrc = ptab[:, 0] + bp[0]
    pdst = ptab[:, 1]

    return _make_sc_final()(psrc, pdst, src, dst)


# speedup vs baseline: 1.2299x; 1.0129x over previous
import functools

import jax
import jax.numpy as jnp
from jax import lax
from jax.experimental import pallas as pl
from jax.experimental.pallas import tpu as pltpu
from jax.experimental.pallas import tpu_sc as plsc

N = 10000
E = 320000
NC = 2
NS = 16
NW = NC * NS
NPAD = 10240
SLAB = NPAD // NS
EPW = E // NW
C = 80
NCHUNK = EPW // C
C3 = 2000
R1 = 144
R2 = 80

_f32 = jnp.float32


def _tc1_body(x_ref, w1_ref, al_ref, ar_ref, srcdata_ref, ert_ref):
    feat = jnp.dot(x_ref[...], w1_ref[...], preferred_element_type=_f32)
    el = jnp.dot(feat, al_ref[...], preferred_element_type=_f32)
    er = jnp.dot(feat, ar_ref[...], preferred_element_type=_f32)
    srcdata_ref[:, 0:16] = el
    srcdata_ref[:, 16:R1] = feat
    ert_ref[...] = er


def _tc2_body(acc_ref, b1_ref, rexp_ref, w2_ref, al2_ref, ar2_ref,
              srcdata2_ref, er2t_ref):
    acc = acc_ref[0] + acc_ref[1]
    dn = jnp.dot(acc[:, 0:16], rexp_ref[...], preferred_element_type=_f32)
    x = jnp.maximum(acc[:, 16:R1] / (dn + 1e-9) + b1_ref[...], 0.0)
    feat2 = jnp.dot(x, w2_ref[...], preferred_element_type=_f32)
    el2 = jnp.dot(feat2, al2_ref[...], preferred_element_type=_f32)
    er2 = jnp.dot(feat2, ar2_ref[...], preferred_element_type=_f32)
    srcdata2_ref[:, 0:16] = el2
    srcdata2_ref[:, 16:R2] = feat2
    er2t_ref[...] = er2


def _tc3_body(acc_ref, b2_ref, rexp2_ref, wp16_ref, ptab_ref):
    acc = acc_ref[0] + acc_ref[1]
    dn = jnp.dot(acc[:, 0:16], rexp2_ref[...], preferred_element_type=_f32)
    x2 = jnp.maximum(acc[:, 16:R2] / (dn + 1e-9) + b2_ref[...], 0.0)
    ptab_ref[...] = jnp.dot(x2, wp16_ref[...], preferred_element_type=_f32)


_BN = 1000
_GRID = N // _BN


def _tc1(features, W1, Al16, Ar16):
    return pl.pallas_call(
        _tc1_body,
        grid=(_GRID,),
        in_specs=[
            pl.BlockSpec((_BN, 128), lambda i: (i, 0)),
            pl.BlockSpec((128, 128), lambda i: (0, 0)),
            pl.BlockSpec((128, 16), lambda i: (0, 0)),
            pl.BlockSpec((128, 16), lambda i: (0, 0)),
        ],
        out_specs=[
            pl.BlockSpec((_BN, R1), lambda i: (i, 0)),
            pl.BlockSpec((_BN, 16), lambda i: (i, 0)),
        ],
        out_shape=[
            jax.ShapeDtypeStruct((N, R1), _f32),
            jax.ShapeDtypeStruct((N, 16), _f32),
        ],
    )(features, W1, Al16, Ar16)


def _tc2(acc1, b1r, Rexp, W2, Al2, Ar2):
    return pl.pallas_call(
        _tc2_body,
        grid=(_GRID,),
        in_specs=[
            pl.BlockSpec((2, _BN, R1), lambda i: (0, i, 0)),
            pl.BlockSpec((1, 128), lambda i: (0, 0)),
            pl.BlockSpec((16, 128), lambda i: (0, 0)),
            pl.BlockSpec((128, 64), lambda i: (0, 0)),
            pl.BlockSpec((64, 16), lambda i: (0, 0)),
            pl.BlockSpec((64, 16), lambda i: (0, 0)),
        ],
        out_specs=[
            pl.BlockSpec((_BN, R2), lambda i: (i, 0)),
            pl.BlockSpec((_BN, 16), lambda i: (i, 0)),
        ],
        out_shape=[
            jax.ShapeDtypeStruct((N, R2), _f32),
            jax.ShapeDtypeStruct((N, 16), _f32),
        ],
    )(acc1, b1r, Rexp, W2, Al2, Ar2)


def _tc3(acc2, b2r, Rexp2, Wp16):
    return pl.pallas_call(
        _tc3_body,
        grid=(_GRID,),
        in_specs=[
            pl.BlockSpec((2, _BN, R2), lambda i: (0, i, 0)),
            pl.BlockSpec((1, 64), lambda i: (0, 0)),
            pl.BlockSpec((16, 64), lambda i: (0, 0)),
            pl.BlockSpec((64, 16), lambda i: (0, 0)),
        ],
        out_specs=pl.BlockSpec((_BN, 16), lambda i: (i, 0)),
        out_shape=jax.ShapeDtypeStruct((N, 16), _f32),
    )(acc2, b2r, Rexp2, Wp16)


BLK = 25


def _make_sc_conv(R, nhchunks, per_head, nbuf, ec, unroll):
    nchunk = EPW // ec
    nblk = nchunk // BLK
    mesh = plsc.VectorSubcoreMesh(core_axis_name="c", subcore_axis_name="s")

    @functools.partial(
        pl.kernel,
        out_type=jax.ShapeDtypeStruct((NC, NPAD, R), _f32),
        mesh=mesh,
        compiler_params=pltpu.CompilerParams(use_tc_tiling_on_sc=False),
        scratch_types=[
            pltpu.VMEM_SHARED((NPAD, R), _f32),
            pltpu.VMEM((BLK, ec), jnp.int32),
            pltpu.VMEM((BLK, ec), jnp.int32),
        ]
        + [pltpu.VMEM((ec, R), _f32) for _ in range(nbuf)]
        + [pltpu.VMEM((ec, 16), _f32) for _ in range(nbuf)]
        + [pltpu.SemaphoreType.DMA for _ in range(3 * nbuf)],
    )
    def sc_conv(srcdata_hbm, ert_hbm, src3_hbm, dst3_hbm, zrow_hbm, out_hbm,
                accum, src_buf, dst_buf, *bufs):
        rows = bufs[0:nbuf]
        erb = bufs[nbuf:2 * nbuf]
        rsem = bufs[2 * nbuf:3 * nbuf]
        esem = bufs[3 * nbuf:4 * nbuf]
        ssem = bufs[4 * nbuf:5 * nbuf]
        c = lax.axis_index("c")
        s = lax.axis_index("s")
        wid = s * NC + c
        slab = s * SLAB
        pltpu.sync_copy(zrow_hbm, rows[0])
        for k in range(SLAB // ec):
            pltpu.sync_copy(rows[0], accum.at[pl.ds(slab + k * ec, ec)])
        rem = SLAB % ec
        if rem:
            pltpu.sync_copy(rows[0].at[pl.ds(0, rem)],
                            accum.at[pl.ds(slab + (SLAB // ec) * ec, rem)])
        plsc.subcore_barrier()

        def issue(kc, b):
            pltpu.async_copy(srcdata_hbm.at[src_buf.at[kc]], rows[b], rsem[b])
            pltpu.async_copy(ert_hbm.at[dst_buf.at[kc]], erb[b], esem[b])

        def wait_scatter(kc, b):
            pltpu.make_async_copy(rows[b], accum.at[dst_buf.at[kc]],
                                  ssem[b]).wait()

        def process(kc, b):
            pltpu.make_async_copy(srcdata_hbm.at[src_buf.at[kc]],
                                  rows[b], rsem[b]).wait()
            pltpu.make_async_copy(ert_hbm.at[dst_buf.at[kc]],
                                  erb[b], esem[b]).wait()

            def edgeu(j, cy):
                i0 = j * unroll
                ts = []
                for u in range(unroll):
                    i = i0 + u
                    v = rows[b][i, 0:16] + erb[b][i, 0:16]
                    t = jnp.exp(jnp.maximum(v, 0.2 * v))
                    rows[b][i, 0:16] = t
                    ts.append(t)
                for u in range(unroll):
                    i = i0 + u
                    t = ts[u]
                    for h in range(nhchunks):
                        g = t[h if per_head else 0]
                        sl = pl.ds(16 + 16 * h, 16)
                        rows[b][i, sl] = rows[b][i, sl] * g
                return cy

            lax.fori_loop(0, ec // unroll, edgeu, 0)
            pltpu.async_copy(rows[b], accum.at[dst_buf.at[kc]], ssem[b],
                             add=True)

        def block(bi, carry):
            pltpu.sync_copy(src3_hbm.at[wid, pl.ds(bi * BLK, BLK)], src_buf)
            pltpu.sync_copy(dst3_hbm.at[wid, pl.ds(bi * BLK, BLK)], dst_buf)
            for b in range(nbuf):
                issue(b, b)
            for kc in range(BLK):
                process(kc, kc % nbuf)
                j = kc + nbuf - 1
                if kc >= 1 and j < BLK:
                    wait_scatter(kc - 1, (kc - 1) % nbuf)
                    issue(j, (kc - 1) % nbuf)
            for kc in range(BLK - nbuf, BLK):
                wait_scatter(kc, kc % nbuf)
            return carry

        lax.fori_loop(0, nblk, block, 0)

        plsc.subcore_barrier()
        pltpu.sync_copy(accum.at[pl.ds(slab, SLAB)],
                        out_hbm.at[c, pl.ds(slab, SLAB)])

    return sc_conv


def _make_sc_final():
    mesh = plsc.VectorSubcoreMesh(core_axis_name="c", subcore_axis_name="s")

    @functools.partial(
        pl.kernel,
        out_type=jax.ShapeDtypeStruct((E,), _f32),
        mesh=mesh,
        compiler_params=pltpu.CompilerParams(
            use_tc_tiling_on_sc=False, needs_layout_passes=False),
        scratch_types=[
            pltpu.VMEM((N,), _f32),
            pltpu.VMEM((N,), _f32),
            pltpu.VMEM((C3,), jnp.int32),
            pltpu.VMEM((C3,), jnp.int32),
            pltpu.VMEM((C3,), _f32),
        ],
    )
    def sc_final(psrc_hbm, pdst_hbm, src_hbm, dst_hbm, out_hbm,
                 pst, pdt, idxs, idxd, ob):
        c = lax.axis_index("c")
        s = lax.axis_index("s")
        wid = s * NC + c
        pltpu.sync_copy(psrc_hbm, pst)
        pltpu.sync_copy(pdst_hbm, pdt)
        ebase = wid * EPW

        def chunk(kc, carry):
            base = ebase + kc * C3
            pltpu.sync_copy(src_hbm.at[pl.ds(base, C3)], idxs)
            pltpu.sync_copy(dst_hbm.at[pl.ds(base, C3)], idxd)

            def grp(j, cy):
                a = plsc.load_gather(pst, [idxs[pl.ds(j * 16, 16)]])
                b = plsc.load_gather(pdt, [idxd[pl.ds(j * 16, 16)]])
                z = a + b
                ob[pl.ds(j * 16, 16)] = 1.0 / (1.0 + jnp.exp(-z))
                return cy

            lax.fori_loop(0, C3 // 16, grp, 0)
            pltpu.sync_copy(ob, out_hbm.at[pl.ds(base, C3)])
            return carry

        lax.fori_loop(0, EPW // C3, chunk, 0)

    return sc_final


def kernel(features, edge_index, edge_type, W1, attn_l1, attn_r1, b1,
           W2, attn_l2, attn_r2, b2, Wp, bp):
    del edge_type
    src = edge_index[0]
    dst = edge_index[1]

    idx128 = jnp.arange(128)
    Al16 = jnp.zeros((128, 16), _f32).at[idx128, idx128 // 16].set(
        attn_l1.reshape(128))
    Ar16 = jnp.zeros((128, 16), _f32).at[idx128, idx128 // 16].set(
        attn_r1.reshape(128))
    Rexp = (jnp.arange(16)[:, None] == (idx128[None, :] // 16)).astype(_f32)
    Al2 = jnp.zeros((64, 16), _f32).at[:, 0].set(attn_l2[0])
    Ar2 = jnp.zeros((64, 16), _f32).at[:, 0].set(attn_r2[0])
    Rexp2 = jnp.zeros((16, 64), _f32).at[0, :].set(1.0)
    Wp16 = jnp.zeros((64, 16), _f32).at[:, 0].set(Wp[:64, 0]).at[:, 1].set(
        Wp[64:, 0])
    b1r = b1.reshape(1, 128)
    b2r = b2.reshape(1, 64)

    srcdata1, er1t = _tc1(features, W1, Al16, Ar16)

    c1 = 50
    src3a = src.reshape(NW, EPW // c1, c1)
    dst3a = dst.reshape(NW, EPW // c1, c1)
    zrow1 = jnp.zeros((c1, R1), _f32)
    acc1 = _make_sc_conv(R1, 8, True, 3, c1, 5)(srcdata1, er1t, src3a, dst3a,
                                                zrow1)

    srcdata2, er2t = _tc2(acc1, b1r, Rexp, W2, Al2, Ar2)

    c2 = 80
    src3b = src.reshape(NW, EPW // c2, c2)
    dst3b = dst.reshape(NW, EPW // c2, c2)
    zrow2 = jnp.zeros((c2, R2), _f32)
    acc2 = _make_sc_conv(R2, 4, False, 4, c2, 4)(srcdata2, er2t, src3b, dst3b,
                                                 zrow2)

    ptab = _tc3(acc2, b2r, Rexp2, Wp16)
    psrc = ptab[:, 0] + bp[0]
    pdst = ptab[:, 1]

    return _make_sc_final()(psrc, pdst, src, dst)


# speedup vs baseline: 1.2898x; 1.0487x over previous
import functools

import jax
import jax.numpy as jnp
from jax import lax
from jax.experimental import pallas as pl
from jax.experimental.pallas import tpu as pltpu
from jax.experimental.pallas import tpu_sc as plsc

N = 10000
E = 320000
NC = 2
NS = 16
NW = NC * NS
NPAD = 10240
SLAB = NPAD // NS
EPW = E // NW
C = 80
NCHUNK = EPW // C
C3 = 2000
R1 = 144
R2 = 80

_f32 = jnp.float32


def _tc1_body(x_ref, w1_ref, al_ref, ar_ref, srcdata_ref, ert_ref):
    feat = jnp.dot(x_ref[...], w1_ref[...], preferred_element_type=_f32)
    el = jnp.dot(feat, al_ref[...], preferred_element_type=_f32)
    er = jnp.dot(feat, ar_ref[...], preferred_element_type=_f32)
    srcdata_ref[:, 0:16] = el
    srcdata_ref[:, 16:R1] = feat
    ert_ref[...] = er


def _tc2_body(acc_ref, b1_ref, rexp_ref, w2_ref, al2_ref, ar2_ref,
              srcdata2_ref, er2t_ref):
    acc = acc_ref[0] + acc_ref[1]
    dn = jnp.dot(acc[:, 0:16], rexp_ref[...], preferred_element_type=_f32)
    x = jnp.maximum(acc[:, 16:R1] / (dn + 1e-9) + b1_ref[...], 0.0)
    feat2 = jnp.dot(x, w2_ref[...], preferred_element_type=_f32)
    el2 = jnp.dot(feat2, al2_ref[...], preferred_element_type=_f32)
    er2 = jnp.dot(feat2, ar2_ref[...], preferred_element_type=_f32)
    srcdata2_ref[:, 0:16] = el2
    srcdata2_ref[:, 16:R2] = feat2
    er2t_ref[...] = er2


def _tc3_body(acc_ref, b2_ref, rexp2_ref, wp16_ref, ptab_ref):
    acc = acc_ref[0] + acc_ref[1]
    dn = jnp.dot(acc[:, 0:16], rexp2_ref[...], preferred_element_type=_f32)
    x2 = jnp.maximum(acc[:, 16:R2] / (dn + 1e-9) + b2_ref[...], 0.0)
    ptab_ref[...] = jnp.dot(x2, wp16_ref[...], preferred_element_type=_f32)


_BN = 1000
_GRID = N // _BN


def _tc1(features, W1, Al16, Ar16):
    return pl.pallas_call(
        _tc1_body,
        grid=(_GRID,),
        in_specs=[
            pl.BlockSpec((_BN, 128), lambda i: (i, 0)),
            pl.BlockSpec((128, 128), lambda i: (0, 0)),
            pl.BlockSpec((128, 16), lambda i: (0, 0)),
            pl.BlockSpec((128, 16), lambda i: (0, 0)),
        ],
        out_specs=[
            pl.BlockSpec((_BN, R1), lambda i: (i, 0)),
            pl.BlockSpec((_BN, 16), lambda i: (i, 0)),
        ],
        out_shape=[
            jax.ShapeDtypeStruct((N, R1), _f32),
            jax.ShapeDtypeStruct((N, 16), _f32),
        ],
    )(features, W1, Al16, Ar16)


def _tc2(acc1, b1r, Rexp, W2, Al2, Ar2):
    return pl.pallas_call(
        _tc2_body,
        grid=(_GRID,),
        in_specs=[
            pl.BlockSpec((2, _BN, R1), lambda i: (0, i, 0)),
            pl.BlockSpec((1, 128), lambda i: (0, 0)),
            pl.BlockSpec((16, 128), lambda i: (0, 0)),
            pl.BlockSpec((128, 64), lambda i: (0, 0)),
            pl.BlockSpec((64, 16), lambda i: (0, 0)),
            pl.BlockSpec((64, 16), lambda i: (0, 0)),
        ],
        out_specs=[
            pl.BlockSpec((_BN, R2), lambda i: (i, 0)),
            pl.BlockSpec((_BN, 16), lambda i: (i, 0)),
        ],
        out_shape=[
            jax.ShapeDtypeStruct((N, R2), _f32),
            jax.ShapeDtypeStruct((N, 16), _f32),
        ],
    )(acc1, b1r, Rexp, W2, Al2, Ar2)


def _tc3(acc2, b2r, Rexp2, Wp16):
    return pl.pallas_call(
        _tc3_body,
        grid=(_GRID,),
        in_specs=[
            pl.BlockSpec((2, _BN, R2), lambda i: (0, i, 0)),
            pl.BlockSpec((1, 64), lambda i: (0, 0)),
            pl.BlockSpec((16, 64), lambda i: (0, 0)),
            pl.BlockSpec((64, 16), lambda i: (0, 0)),
        ],
        out_specs=pl.BlockSpec((_BN, 16), lambda i: (i, 0)),
        out_shape=jax.ShapeDtypeStruct((N, 16), _f32),
    )(acc2, b2r, Rexp2, Wp16)


BLK = 25


def _make_sc_conv(R, nhchunks, per_head, nbuf, ec, unroll):
    nchunk = EPW // ec
    nblk = nchunk // BLK
    mesh = plsc.VectorSubcoreMesh(core_axis_name="c", subcore_axis_name="s")

    @functools.partial(
        pl.kernel,
        out_type=jax.ShapeDtypeStruct((NC, NPAD, R), _f32),
        mesh=mesh,
        compiler_params=pltpu.CompilerParams(use_tc_tiling_on_sc=False),
        scratch_types=[
            pltpu.VMEM_SHARED((NPAD, R), _f32),
            pltpu.VMEM((BLK, ec), jnp.int32),
            pltpu.VMEM((BLK, ec), jnp.int32),
        ]
        + [pltpu.VMEM((ec, R), _f32) for _ in range(nbuf)]
        + [pltpu.VMEM((ec, 16), _f32) for _ in range(nbuf)]
        + [pltpu.SemaphoreType.DMA for _ in range(3 * nbuf)],
    )
    def sc_conv(srcdata_hbm, ert_hbm, src3_hbm, dst3_hbm, zrow_hbm, out_hbm,
                accum, src_buf, dst_buf, *bufs):
        rows = bufs[0:nbuf]
        erb = bufs[nbuf:2 * nbuf]
        rsem = bufs[2 * nbuf:3 * nbuf]
        esem = bufs[3 * nbuf:4 * nbuf]
        ssem = bufs[4 * nbuf:5 * nbuf]
        c = lax.axis_index("c")
        s = lax.axis_index("s")
        wid = s * NC + c
        slab = s * SLAB
        pltpu.sync_copy(zrow_hbm, rows[0])
        for k in range(SLAB // ec):
            pltpu.sync_copy(rows[0], accum.at[pl.ds(slab + k * ec, ec)])
        rem = SLAB % ec
        if rem:
            pltpu.sync_copy(rows[0].at[pl.ds(0, rem)],
                            accum.at[pl.ds(slab + (SLAB // ec) * ec, rem)])
        plsc.subcore_barrier()

        def issue(kc, b):
            pltpu.async_copy(srcdata_hbm.at[src_buf.at[kc]], rows[b], rsem[b])
            pltpu.async_copy(ert_hbm.at[dst_buf.at[kc]], erb[b], esem[b])

        def wait_scatter(kc, b):
            pltpu.make_async_copy(rows[b], accum.at[dst_buf.at[kc]],
                                  ssem[b]).wait()

        def process(kc, b):
            pltpu.make_async_copy(srcdata_hbm.at[src_buf.at[kc]],
                                  rows[b], rsem[b]).wait()
            pltpu.make_async_copy(ert_hbm.at[dst_buf.at[kc]],
                                  erb[b], esem[b]).wait()

            def edgeu(j, cy):
                i0 = j * unroll
                ts = []
                for u in range(unroll):
                    i = i0 + u
                    v = rows[b][i, 0:16] + erb[b][i, 0:16]
                    t = jnp.exp(jnp.maximum(v, 0.2 * v))
                    rows[b][i, 0:16] = t
                    ts.append(t)
                for u in range(unroll):
                    i = i0 + u
                    t = ts[u]
                    for h in range(nhchunks):
                        g = t[h if per_head else 0]
                        sl = pl.ds(16 + 16 * h, 16)
                        rows[b][i, sl] = rows[b][i, sl] * g
                return cy

            lax.fori_loop(0, ec // unroll, edgeu, 0)
            pltpu.async_copy(rows[b], accum.at[dst_buf.at[kc]], ssem[b],
                             add=True)

        def block(bi, carry):
            pltpu.sync_copy(src3_hbm.at[wid, pl.ds(bi * BLK, BLK)], src_buf)
            pltpu.sync_copy(dst3_hbm.at[wid, pl.ds(bi * BLK, BLK)], dst_buf)
            for b in range(nbuf):
                issue(b, b)
            for kc in range(BLK):
                process(kc, kc % nbuf)
                j = kc + nbuf - 1
                if kc >= 1 and j < BLK:
                    wait_scatter(kc - 1, (kc - 1) % nbuf)
                    issue(j, (kc - 1) % nbuf)
            for kc in range(BLK - nbuf, BLK):
                wait_scatter(kc, kc % nbuf)
            return carry

        lax.fori_loop(0, nblk, block, 0)

        plsc.subcore_barrier()
        pltpu.sync_copy(accum.at[pl.ds(slab, SLAB)],
                        out_hbm.at[c, pl.ds(slab, SLAB)])

    return sc_conv


def _make_sc_final():
    mesh = plsc.VectorSubcoreMesh(core_axis_name="c", subcore_axis_name="s")

    @functools.partial(
        pl.kernel,
        out_type=jax.ShapeDtypeStruct((E,), _f32),
        mesh=mesh,
        compiler_params=pltpu.CompilerParams(
            use_tc_tiling_on_sc=False, needs_layout_passes=False),
        scratch_types=[
            pltpu.VMEM((N,), _f32),
            pltpu.VMEM((N,), _f32),
            pltpu.VMEM((C3,), jnp.int32),
            pltpu.VMEM((C3,), jnp.int32),
            pltpu.VMEM((C3,), _f32),
        ],
    )
    def sc_final(psrc_hbm, pdst_hbm, src_hbm, dst_hbm, out_hbm,
                 pst, pdt, idxs, idxd, ob):
        c = lax.axis_index("c")
        s = lax.axis_index("s")
        wid = s * NC + c
        pltpu.sync_copy(psrc_hbm, pst)
        pltpu.sync_copy(pdst_hbm, pdt)
        ebase = wid * EPW

        def chunk(kc, carry):
            base = ebase + kc * C3
            pltpu.sync_copy(src_hbm.at[pl.ds(base, C3)], idxs)
            pltpu.sync_copy(dst_hbm.at[pl.ds(base, C3)], idxd)

            def grp(j, cy):
                a = plsc.load_gather(pst, [idxs[pl.ds(j * 16, 16)]])
                b = plsc.load_gather(pdt, [idxd[pl.ds(j * 16, 16)]])
                z = a + b
                ob[pl.ds(j * 16, 16)] = 1.0 / (1.0 + jnp.exp(-z))
                return cy

            lax.fori_loop(0, C3 // 16, grp, 0)
            pltpu.sync_copy(ob, out_hbm.at[pl.ds(base, C3)])
            return carry

        lax.fori_loop(0, EPW // C3, chunk, 0)

    return sc_final


def kernel(features, edge_index, edge_type, W1, attn_l1, attn_r1, b1,
           W2, attn_l2, attn_r2, b2, Wp, bp):
    del edge_type
    src = edge_index[0]
    dst = edge_index[1]

    idx128 = jnp.arange(128)
    Al16 = jnp.zeros((128, 16), _f32).at[idx128, idx128 // 16].set(
        attn_l1.reshape(128))
    Ar16 = jnp.zeros((128, 16), _f32).at[idx128, idx128 // 16].set(
        attn_r1.reshape(128))
    Rexp = (jnp.arange(16)[:, None] == (idx128[None, :] // 16)).astype(_f32)
    Al2 = jnp.zeros((64, 16), _f32).at[:, 0].set(attn_l2[0])
    Ar2 = jnp.zeros((64, 16), _f32).at[:, 0].set(attn_r2[0])
    Rexp2 = jnp.zeros((16, 64), _f32).at[0, :].set(1.0)
    Wp16 = jnp.zeros((64, 16), _f32).at[:, 0].set(Wp[:64, 0]).at[:, 1].set(
        Wp[64:, 0])
    b1r = b1.reshape(1, 128)
    b2r = b2.reshape(1, 64)

    srcdata1, er1t = _tc1(features, W1, Al16, Ar16)

    c1 = 40
    src3a = src.reshape(NW, EPW // c1, c1)
    dst3a = dst.reshape(NW, EPW // c1, c1)
    zrow1 = jnp.zeros((c1, R1), _f32)
    acc1 = _make_sc_conv(R1, 8, True, 4, c1, 5)(srcdata1, er1t, src3a, dst3a,
                                                zrow1)

    srcdata2, er2t = _tc2(acc1, b1r, Rexp, W2, Al2, Ar2)

    c2 = 80
    src3b = src.reshape(NW, EPW // c2, c2)
    dst3b = dst.reshape(NW, EPW // c2, c2)
    zrow2 = jnp.zeros((c2, R2), _f32)
    acc2 = _make_sc_conv(R2, 4, False, 6, c2, 4)(srcdata2, er2t, src3b, dst3b,
                                                 zrow2)

    ptab = _tc3(acc2, b2r, Rexp2, Wp16)
    psrc = ptab[:, 0] + bp[0]
    pdst = ptab[:, 1]

    return _make_sc_final()(psrc, pdst, src, dst)


# speedup vs baseline: 1.3036x; 1.0107x over previous
import functools

import jax
import jax.numpy as jnp
from jax import lax
from jax.experimental import pallas as pl
from jax.experimental.pallas import tpu as pltpu
from jax.experimental.pallas import tpu_sc as plsc

N = 10000
E = 320000
NC = 2
NS = 16
NW = NC * NS
NPAD = 10240
SLAB = NPAD // NS
EPW = E // NW
C = 80
NCHUNK = EPW // C
C3 = 2000
R1 = 144
R2 = 80

_f32 = jnp.float32


def _tc1_body(x_ref, w1_ref, al_ref, ar_ref, srcdata_ref, ert_ref):
    feat = jnp.dot(x_ref[...], w1_ref[...], preferred_element_type=_f32)
    el = jnp.dot(feat, al_ref[...], preferred_element_type=_f32)
    er = jnp.dot(feat, ar_ref[...], preferred_element_type=_f32)
    srcdata_ref[:, 0:16] = el
    srcdata_ref[:, 16:R1] = feat
    ert_ref[...] = er


def _tc2_body(acc_ref, b1_ref, rexp_ref, w2_ref, al2_ref, ar2_ref,
              srcdata2_ref, er2t_ref):
    acc = acc_ref[0] + acc_ref[1]
    dn = jnp.dot(acc[:, 0:16], rexp_ref[...], preferred_element_type=_f32)
    x = jnp.maximum(acc[:, 16:R1] / (dn + 1e-9) + b1_ref[...], 0.0)
    feat2 = jnp.dot(x, w2_ref[...], preferred_element_type=_f32)
    el2 = jnp.dot(feat2, al2_ref[...], preferred_element_type=_f32)
    er2 = jnp.dot(feat2, ar2_ref[...], preferred_element_type=_f32)
    srcdata2_ref[:, 0:16] = el2
    srcdata2_ref[:, 16:R2] = feat2
    er2t_ref[...] = er2


def _tc3_body(acc_ref, b2_ref, rexp2_ref, wp16_ref, ptab_ref):
    acc = acc_ref[0] + acc_ref[1]
    dn = jnp.dot(acc[:, 0:16], rexp2_ref[...], preferred_element_type=_f32)
    x2 = jnp.maximum(acc[:, 16:R2] / (dn + 1e-9) + b2_ref[...], 0.0)
    ptab_ref[...] = jnp.dot(x2, wp16_ref[...], preferred_element_type=_f32)


_BN = 1000
_GRID = N // _BN


def _tc1(features, W1, Al16, Ar16):
    return pl.pallas_call(
        _tc1_body,
        grid=(_GRID,),
        in_specs=[
            pl.BlockSpec((_BN, 128), lambda i: (i, 0)),
            pl.BlockSpec((128, 128), lambda i: (0, 0)),
            pl.BlockSpec((128, 16), lambda i: (0, 0)),
            pl.BlockSpec((128, 16), lambda i: (0, 0)),
        ],
        out_specs=[
            pl.BlockSpec((_BN, R1), lambda i: (i, 0)),
            pl.BlockSpec((_BN, 16), lambda i: (i, 0)),
        ],
        out_shape=[
            jax.ShapeDtypeStruct((N, R1), _f32),
            jax.ShapeDtypeStruct((N, 16), _f32),
        ],
    )(features, W1, Al16, Ar16)


def _tc2(acc1, b1r, Rexp, W2, Al2, Ar2):
    return pl.pallas_call(
        _tc2_body,
        grid=(_GRID,),
        in_specs=[
            pl.BlockSpec((2, _BN, R1), lambda i: (0, i, 0)),
            pl.BlockSpec((1, 128), lambda i: (0, 0)),
            pl.BlockSpec((16, 128), lambda i: (0, 0)),
            pl.BlockSpec((128, 64), lambda i: (0, 0)),
            pl.BlockSpec((64, 16), lambda i: (0, 0)),
            pl.BlockSpec((64, 16), lambda i: (0, 0)),
        ],
        out_specs=[
            pl.BlockSpec((_BN, R2), lambda i: (i, 0)),
            pl.BlockSpec((_BN, 16), lambda i: (i, 0)),
        ],
        out_shape=[
            jax.ShapeDtypeStruct((N, R2), _f32),
            jax.ShapeDtypeStruct((N, 16), _f32),
        ],
    )(acc1, b1r, Rexp, W2, Al2, Ar2)


def _tc3(acc2, b2r, Rexp2, Wp16):
    return pl.pallas_call(
        _tc3_body,
        grid=(_GRID,),
        in_specs=[
            pl.BlockSpec((2, _BN, R2), lambda i: (0, i, 0)),
            pl.BlockSpec((1, 64), lambda i: (0, 0)),
            pl.BlockSpec((16, 64), lambda i: (0, 0)),
            pl.BlockSpec((64, 16), lambda i: (0, 0)),
        ],
        out_specs=pl.BlockSpec((_BN, 16), lambda i: (i, 0)),
        out_shape=jax.ShapeDtypeStruct((N, 16), _f32),
    )(acc2, b2r, Rexp2, Wp16)


BLK = 25


def _make_sc_conv(R, nhchunks, per_head, nbuf, ec, unroll):
    nchunk = EPW // ec
    nblk = nchunk // BLK
    mesh = plsc.VectorSubcoreMesh(core_axis_name="c", subcore_axis_name="s")

    @functools.partial(
        pl.kernel,
        out_type=jax.ShapeDtypeStruct((NC, NPAD, R), _f32),
        mesh=mesh,
        compiler_params=pltpu.CompilerParams(use_tc_tiling_on_sc=False),
        scratch_types=[
            pltpu.VMEM_SHARED((NPAD, R), _f32),
            pltpu.VMEM((BLK, ec), jnp.int32),
            pltpu.VMEM((BLK, ec), jnp.int32),
        ]
        + [pltpu.VMEM((ec, R), _f32) for _ in range(nbuf)]
        + [pltpu.VMEM((ec, 16), _f32) for _ in range(nbuf)]
        + [pltpu.SemaphoreType.DMA for _ in range(3 * nbuf)],
    )
    def sc_conv(srcdata_hbm, ert_hbm, src3_hbm, dst3_hbm, zrow_hbm, out_hbm,
                accum, src_buf, dst_buf, *bufs):
        rows = bufs[0:nbuf]
        erb = bufs[nbuf:2 * nbuf]
        rsem = bufs[2 * nbuf:3 * nbuf]
        esem = bufs[3 * nbuf:4 * nbuf]
        ssem = bufs[4 * nbuf:5 * nbuf]
        c = lax.axis_index("c")
        s = lax.axis_index("s")
        wid = s * NC + c
        slab = s * SLAB
        pltpu.sync_copy(zrow_hbm, rows[0])
        for k in range(SLAB // ec):
            pltpu.sync_copy(rows[0], accum.at[pl.ds(slab + k * ec, ec)])
        rem = SLAB % ec
        if rem:
            pltpu.sync_copy(rows[0].at[pl.ds(0, rem)],
                            accum.at[pl.ds(slab + (SLAB // ec) * ec, rem)])
        plsc.subcore_barrier()

        def issue(kc, b):
            pltpu.async_copy(srcdata_hbm.at[src_buf.at[kc]], rows[b], rsem[b])
            pltpu.async_copy(ert_hbm.at[dst_buf.at[kc]], erb[b], esem[b])

        def wait_scatter(kc, b):
            pltpu.make_async_copy(rows[b], accum.at[dst_buf.at[kc]],
                                  ssem[b]).wait()

        def process(kc, b):
            pltpu.make_async_copy(srcdata_hbm.at[src_buf.at[kc]],
                                  rows[b], rsem[b]).wait()
            pltpu.make_async_copy(ert_hbm.at[dst_buf.at[kc]],
                                  erb[b], esem[b]).wait()

            def edgeu(j, cy):
                i0 = j * unroll
                ts = []
                for u in range(unroll):
                    i = i0 + u
                    v = rows[b][i, 0:16] + erb[b][i, 0:16]
                    t = jnp.exp(jnp.maximum(v, 0.2 * v))
                    rows[b][i, 0:16] = t
                    ts.append(t)
                for u in range(unroll):
                    i = i0 + u
                    t = ts[u]
                    for h in range(nhchunks):
                        g = t[h if per_head else 0]
                        sl = pl.ds(16 + 16 * h, 16)
                        rows[b][i, sl] = rows[b][i, sl] * g
                return cy

            lax.fori_loop(0, ec // unroll, edgeu, 0)
            pltpu.async_copy(rows[b], accum.at[dst_buf.at[kc]], ssem[b],
                             add=True)

        def block(bi, carry):
            pltpu.sync_copy(src3_hbm.at[wid, pl.ds(bi * BLK, BLK)], src_buf)
            pltpu.sync_copy(dst3_hbm.at[wid, pl.ds(bi * BLK, BLK)], dst_buf)
            for b in range(nbuf):
                issue(b, b)
            for kc in range(BLK):
                process(kc, kc % nbuf)
                j = kc + nbuf - 1
                if kc >= 1 and j < BLK:
                    wait_scatter(kc - 1, (kc - 1) % nbuf)
                    issue(j, (kc - 1) % nbuf)
            for kc in range(BLK - nbuf, BLK):
                wait_scatter(kc, kc % nbuf)
            return carry

        lax.fori_loop(0, nblk, block, 0)

        plsc.subcore_barrier()
        pltpu.sync_copy(accum.at[pl.ds(slab, SLAB)],
                        out_hbm.at[c, pl.ds(slab, SLAB)])

    return sc_conv


def _make_sc_final():
    mesh = plsc.VectorSubcoreMesh(core_axis_name="c", subcore_axis_name="s")

    @functools.partial(
        pl.kernel,
        out_type=jax.ShapeDtypeStruct((E,), _f32),
        mesh=mesh,
        compiler_params=pltpu.CompilerParams(
            use_tc_tiling_on_sc=False, needs_layout_passes=False),
        scratch_types=[
            pltpu.VMEM((N,), _f32),
            pltpu.VMEM((N,), _f32),
            pltpu.VMEM((C3,), jnp.int32),
            pltpu.VMEM((C3,), jnp.int32),
            pltpu.VMEM((C3,), _f32),
        ],
    )
    def sc_final(psrc_hbm, pdst_hbm, src_hbm, dst_hbm, out_hbm,
                 pst, pdt, idxs, idxd, ob):
        c = lax.axis_index("c")
        s = lax.axis_index("s")
        wid = s * NC + c
        pltpu.sync_copy(psrc_hbm, pst)
        pltpu.sync_copy(pdst_hbm, pdt)
        ebase = wid * EPW

        def chunk(kc, carry):
            base = ebase + kc * C3
            pltpu.sync_copy(src_hbm.at[pl.ds(base, C3)], idxs)
            pltpu.sync_copy(dst_hbm.at[pl.ds(base, C3)], idxd)

            def grp(j, cy):
                a = plsc.load_gather(pst, [idxs[pl.ds(j * 16, 16)]])
                b = plsc.load_gather(pdt, [idxd[pl.ds(j * 16, 16)]])
                z = a + b
                ob[pl.ds(j * 16, 16)] = 1.0 / (1.0 + jnp.exp(-z))
                return cy

            lax.fori_loop(0, C3 // 16, grp, 0)
            pltpu.sync_copy(ob, out_hbm.at[pl.ds(base, C3)])
            return carry

        lax.fori_loop(0, EPW // C3, chunk, 0)

    return sc_final


def kernel(features, edge_index, edge_type, W1, attn_l1, attn_r1, b1,
           W2, attn_l2, attn_r2, b2, Wp, bp):
    del edge_type
    src = edge_index[0]
    dst = edge_index[1]

    idx128 = jnp.arange(128)
    Al16 = jnp.zeros((128, 16), _f32).at[idx128, idx128 // 16].set(
        attn_l1.reshape(128))
    Ar16 = jnp.zeros((128, 16), _f32).at[idx128, idx128 // 16].set(
        attn_r1.reshape(128))
    Rexp = (jnp.arange(16)[:, None] == (idx128[None, :] // 16)).astype(_f32)
    Al2 = jnp.zeros((64, 16), _f32).at[:, 0].set(attn_l2[0])
    Ar2 = jnp.zeros((64, 16), _f32).at[:, 0].set(attn_r2[0])
    Rexp2 = jnp.zeros((16, 64), _f32).at[0, :].set(1.0)
    Wp16 = jnp.zeros((64, 16), _f32).at[:, 0].set(Wp[:64, 0]).at[:, 1].set(
        Wp[64:, 0])
    b1r = b1.reshape(1, 128)
    b2r = b2.reshape(1, 64)

    srcdata1, er1t = _tc1(features, W1, Al16, Ar16)

    c1 = 40
    src3a = src.reshape(NW, EPW // c1, c1)
    dst3a = dst.reshape(NW, EPW // c1, c1)
    zrow1 = jnp.zeros((c1, R1), _f32)
    acc1 = _make_sc_conv(R1, 8, True, 5, c1, 5)(srcdata1, er1t, src3a, dst3a,
                                                zrow1)

    srcdata2, er2t = _tc2(acc1, b1r, Rexp, W2, Al2, Ar2)

    c2 = 80
    src3b = src.reshape(NW, EPW // c2, c2)
    dst3b = dst.reshape(NW, EPW // c2, c2)
    zrow2 = jnp.zeros((c2, R2), _f32)
    acc2 = _make_sc_conv(R2, 4, False, 8, c2, 4)(srcdata2, er2t, src3b, dst3b,
                                                 zrow2)

    ptab = _tc3(acc2, b2r, Rexp2, Wp16)
    psrc = ptab[:, 0] + bp[0]
    pdst = ptab[:, 1]

    return _make_sc_final()(psrc, pdst, src, dst)
